# baseline (speedup 1.0000x reference)
import functools

import jax
import jax.numpy as jnp
from jax import lax
from jax.experimental import pallas as pl
from jax.experimental.pallas import tpu as pltpu
from jax.experimental.pallas import tpu_sc as plsc

EMBED_DIM = 64
HALF = EMBED_DIM // 2
BATCH = 16384
MODE_PAD = 8
TABLE_ROWS = 16 * MODE_PAD

NC = 2
NS = 16
L = 16
NW = NC * NS
BPW = BATCH // NW
GCH = 128
NG = BPW // GCH

MMR = BATCH // 32
IDXC = 32


def _tc_body(mm_ref, mode_ref, miss_ref, mode_tab_ref, w1t_ref, w2t_ref,
             b_ref, table_ref, idx_ref):
    miss_proj = jnp.dot(miss_ref[...], w1t_ref[...],
                        preferred_element_type=jnp.float32)
    mode_proj = jnp.dot(mode_tab_ref[...], w2t_ref[...],
                        preferred_element_type=jnp.float32)
    table_ref[...] = (miss_proj[:, None, :] + mode_proj[None, :, :]
                      + b_ref[...][None])

    li = lax.broadcasted_iota(jnp.int32, (128, 128), 0)
    ji = lax.broadcasted_iota(jnp.int32, (128, 128), 1)
    w = jnp.right_shift(jnp.full((128, 128), 64, jnp.int32), li % 4)
    P = jnp.where(ji == li // 4, w, 0).astype(jnp.float32)
    G = jnp.dot(mm_ref[...].astype(jnp.float32), P,
                preferred_element_type=jnp.float32)
    idx_ref[...] = G[:, :IDXC].astype(jnp.int32) + mode_ref[...]


def _tc_stage(missing_mask, mode_id, missing_table, mode_table, W, b):
    w1t = W[:, :HALF].T
    w2t = W[:, HALF:].T
    mode_pad = jnp.zeros((MODE_PAD, HALF), jnp.float32).at[:5].set(mode_table)
    mm = missing_mask.astype(jnp.int32).reshape(MMR, 128)
    mode2 = mode_id.astype(jnp.int32).reshape(MMR, IDXC)
    t3, idx2 = pl.pallas_call(
        _tc_body,
        out_shape=(
            jax.ShapeDtypeStruct((16, MODE_PAD, EMBED_DIM), jnp.float32),
            jax.ShapeDtypeStruct((MMR, IDXC), jnp.int32),
        ),
    )(mm, mode2, missing_table, mode_pad, w1t, w2t, b.reshape(1, EMBED_DIM))
    return t3.reshape(TABLE_ROWS, EMBED_DIM), idx2.reshape(NW, NG, GCH)


@functools.cache
def _make_sc_gather():
    mesh = plsc.VectorSubcoreMesh(core_axis_name="c", subcore_axis_name="s")

    @functools.partial(
        pl.kernel,
        mesh=mesh,
        compiler_params=pltpu.CompilerParams(use_tc_tiling_on_sc=False),
        out_type=jax.ShapeDtypeStruct((BATCH, EMBED_DIM), jnp.float32),
        scratch_types=[
            pltpu.VMEM((NG, GCH), jnp.int32),
            pltpu.VMEM((BPW, EMBED_DIM), jnp.float32),
            pltpu.SemaphoreType.DMA,
        ],
    )
    def _sc_gather(idx_hbm, table_hbm, out_hbm, idx_v, rows_v, sem):
        wid = lax.axis_index("s") * NC + lax.axis_index("c")
        base = wid * BPW

        pltpu.sync_copy(idx_hbm.at[wid], idx_v)
        copies = []
        for g in range(NG):
            copies.append(pltpu.async_copy(
                table_hbm.at[idx_v.at[g]], rows_v.at[pl.ds(g * GCH, GCH)],
                sem))
        for c in copies:
            c.wait()
        pltpu.sync_copy(rows_v, out_hbm.at[pl.ds(base, BPW)])

    return _sc_gather


def kernel(missing_mask, mode_id, missing_table, mode_table, W, b):
    table, idx3 = _tc_stage(missing_mask, mode_id, missing_table,
                            mode_table, W, b)
    return _make_sc_gather()(idx3, table)
